# baseline (speedup 1.0000x reference)
import functools

import jax
import jax.numpy as jnp
from jax import lax
from jax.experimental import pallas as pl
from jax.experimental.pallas import tpu as pltpu
from jax.experimental.pallas import tpu_sc as plsc

_B = 1024
_D = 32
_R = 1000
_PACK = 128 // _D
_VP = 1000000 // _PACK

_NC = 2
_NS = 16
_NW = _NC * _NS
_NIDX = 2 * _B
_BPW = _NIDX // _NW
_L = 16

_sc_mesh = plsc.VectorSubcoreMesh(core_axis_name="c", subcore_axis_name="s")


@functools.partial(
    pl.kernel,
    mesh=_sc_mesh,
    out_type=jax.ShapeDtypeStruct((_NIDX, 4 * _D), jnp.float32),
    scratch_types=[
        pltpu.VMEM((_BPW,), jnp.int32),
        pltpu.VMEM((_BPW, 4 * _D), jnp.float32),
        pltpu.SemaphoreType.DMA,
    ],
)
def _sc_gather(idx_hbm, table_hbm, out_hbm, idx_v, rows_v, sem):
    wid = lax.axis_index("s") * _NC + lax.axis_index("c")
    base = wid * _BPW
    pltpu.sync_copy(idx_hbm.at[pl.ds(base, _BPW)], idx_v)
    for j in range(0, _BPW, _L):
        idx_v[pl.ds(j, _L)] = lax.shift_right_logical(idx_v[pl.ds(j, _L)], 2)
    pltpu.async_copy(table_hbm.at[idx_v], rows_v, sem).wait()
    pltpu.sync_copy(rows_v, out_hbm.at[pl.ds(base, _BPW)])


_BB = 256


def _quarter(p, q):
    lo = jnp.where(q == 0, p[:, 0 * _D:1 * _D], p[:, 1 * _D:2 * _D])
    hi = jnp.where(q == 2, p[:, 2 * _D:3 * _D], p[:, 3 * _D:4 * _D])
    return jnp.where(q < 2, lo, hi)


def _tc_distance_body(hp_ref, tp_ref, hq_ref, tq_ref, rel_t_ref, out_ref):
    h = _quarter(hp_ref[...], hq_ref[...])
    t = _quarter(tp_ref[...], tq_ref[...])
    d = h - t
    acc = jnp.abs(d[:, 0:1] + rel_t_ref[0:1, :])
    for k in range(1, _D):
        acc = acc + jnp.abs(d[:, k:k + 1] + rel_t_ref[k:k + 1, :])
    out_ref[...] = -acc


def _tc_distance(rows, quarters, rel_t):
    nblk = _B // _BB
    return pl.pallas_call(
        _tc_distance_body,
        grid=(nblk,),
        in_specs=[
            pl.BlockSpec((_BB, 4 * _D), lambda i: (i, 0)),
            pl.BlockSpec((_BB, 4 * _D), lambda i: (i + nblk, 0)),
            pl.BlockSpec((_BB, 1), lambda i: (i, 0)),
            pl.BlockSpec((_BB, 1), lambda i: (i + nblk, 0)),
            pl.BlockSpec((_D, _R), lambda i: (0, 0)),
        ],
        out_specs=pl.BlockSpec((_BB, _R), lambda i: (i, 0)),
        out_shape=jax.ShapeDtypeStruct((_B, _R), jnp.float32),
    )(rows, rows, quarters, quarters, rel_t)


def kernel(heads, tails, entity_emb, relation_emb):
    idx = jnp.concatenate([heads, tails]).astype(jnp.int32)
    table_packed = entity_emb.reshape(_VP, 4 * _D)
    rows = _sc_gather(idx, table_packed)
    quarters = (idx & 3).reshape(_NIDX, 1)
    return _tc_distance(rows, quarters, relation_emb.T)


# speedup vs baseline: 1.6269x; 1.6269x over previous
import functools

import jax
import jax.numpy as jnp
from jax import lax
from jax.experimental import pallas as pl
from jax.experimental.pallas import tpu as pltpu
from jax.experimental.pallas import tpu_sc as plsc

_B = 1024
_D = 32
_R = 1000
_PACK = 128 // _D
_VP = 1000000 // _PACK

_NC = 2
_NS = 16
_NW = _NC * _NS
_NIDX = 2 * _B
_BPW = _NIDX // _NW
_L = 16

_sc_mesh = plsc.VectorSubcoreMesh(core_axis_name="c", subcore_axis_name="s")


_CHUNK = 16


@functools.partial(
    pl.kernel,
    mesh=_sc_mesh,
    out_type=jax.ShapeDtypeStruct((_NIDX, _D), jnp.float32),
    scratch_types=[
        pltpu.VMEM((_BPW,), jnp.int32),
        pltpu.VMEM((_BPW, _D), jnp.float32),
        pltpu.SemaphoreType.DMA,
    ],
)
def _sc_gather(idx_hbm, table_hbm, out_hbm, idx_v, rows_v, sem):
    wid = lax.axis_index("s") * _NC + lax.axis_index("c")
    base = wid * _BPW
    pltpu.sync_copy(idx_hbm.at[pl.ds(base, _BPW)], idx_v)
    for c in range(0, _BPW, _CHUNK):
        v = idx_v[pl.ds(c, _CHUNK)]
        cps = [
            pltpu.async_copy(table_hbm.at[v[j]], rows_v.at[c + j], sem)
            for j in range(_CHUNK)
        ]
        for cp in cps:
            cp.wait()
    pltpu.sync_copy(rows_v, out_hbm.at[pl.ds(base, _BPW)])


_BB = 256


def _tc_distance_body(h_ref, t_ref, rel_t_ref, out_ref):
    d = h_ref[...] - t_ref[...]
    acc = jnp.abs(d[:, 0:1] + rel_t_ref[0:1, :])
    for k in range(1, _D):
        acc = acc + jnp.abs(d[:, k:k + 1] + rel_t_ref[k:k + 1, :])
    out_ref[...] = -acc


def _tc_distance(rows, rel_t):
    nblk = _B // _BB
    return pl.pallas_call(
        _tc_distance_body,
        grid=(nblk,),
        in_specs=[
            pl.BlockSpec((_BB, _D), lambda i: (i, 0)),
            pl.BlockSpec((_BB, _D), lambda i: (i + nblk, 0)),
            pl.BlockSpec((_D, _R), lambda i: (0, 0)),
        ],
        out_specs=pl.BlockSpec((_BB, _R), lambda i: (i, 0)),
        out_shape=jax.ShapeDtypeStruct((_B, _R), jnp.float32),
    )(rows, rows, rel_t)


def kernel(heads, tails, entity_emb, relation_emb):
    idx = jnp.concatenate([heads, tails]).astype(jnp.int32)
    rows = _sc_gather(idx, entity_emb)
    return _tc_distance(rows, relation_emb.T)


# speedup vs baseline: 1.6722x; 1.0279x over previous
import jax
import jax.numpy as jnp
from jax import lax
from jax.experimental import pallas as pl
from jax.experimental.pallas import tpu as pltpu

_B = 1024
_D = 32
_R = 1000
_NIDX = 2 * _B


def _tc_gather_body(idx_s, table_ref, out_ref, sem):
    def fire(j, _):
        pltpu.make_async_copy(
            table_ref.at[pl.ds(idx_s[j], 1)], out_ref.at[pl.ds(j, 1)], sem
        ).start()
        return 0

    lax.fori_loop(0, _NIDX, fire, 0, unroll=4)

    def drain(j, _):
        pltpu.make_async_copy(
            table_ref.at[pl.ds(0, 1)], out_ref.at[pl.ds(j, 1)], sem
        ).wait()
        return 0

    lax.fori_loop(0, _NIDX, drain, 0, unroll=4)


def _tc_gather(idx, table):
    return pl.pallas_call(
        _tc_gather_body,
        grid_spec=pltpu.PrefetchScalarGridSpec(
            num_scalar_prefetch=1,
            grid=(1,),
            in_specs=[pl.BlockSpec(memory_space=pl.ANY)],
            out_specs=pl.BlockSpec(memory_space=pltpu.VMEM),
            scratch_shapes=[pltpu.SemaphoreType.DMA],
        ),
        out_shape=jax.ShapeDtypeStruct((_NIDX, _D), jnp.float32),
    )(idx, table)


_BB = 256


def _tc_distance_body(h_ref, t_ref, rel_t_ref, out_ref):
    d = h_ref[...] - t_ref[...]
    acc = jnp.abs(d[:, 0:1] + rel_t_ref[0:1, :])
    for k in range(1, _D):
        acc = acc + jnp.abs(d[:, k:k + 1] + rel_t_ref[k:k + 1, :])
    out_ref[...] = -acc


def _tc_distance(rows, rel_t):
    nblk = _B // _BB
    return pl.pallas_call(
        _tc_distance_body,
        grid=(nblk,),
        in_specs=[
            pl.BlockSpec((_BB, _D), lambda i: (i, 0)),
            pl.BlockSpec((_BB, _D), lambda i: (i + nblk, 0)),
            pl.BlockSpec((_D, _R), lambda i: (0, 0)),
        ],
        out_specs=pl.BlockSpec((_BB, _R), lambda i: (i, 0)),
        out_shape=jax.ShapeDtypeStruct((_B, _R), jnp.float32),
    )(rows, rows, rel_t)


def kernel(heads, tails, entity_emb, relation_emb):
    idx = jnp.concatenate([heads, tails]).astype(jnp.int32)
    rows = _tc_gather(idx, entity_emb)
    return _tc_distance(rows, relation_emb.T)


# speedup vs baseline: 7.9966x; 4.7821x over previous
import functools

import jax
import jax.numpy as jnp
from jax import lax
from jax.experimental import pallas as pl
from jax.experimental.pallas import tpu as pltpu
from jax.experimental.pallas import tpu_sc as plsc

_B = 1024
_D = 32
_R = 1000
_NIDX = 2 * _B

_NC = 2
_NS = 16
_NW = _NC * _NS
_NACT = 16
_BPT = _NIDX // _NACT
_CH = 16

_sc_mesh = plsc.VectorSubcoreMesh(core_axis_name="c", subcore_axis_name="s")


@functools.partial(
    pl.kernel,
    mesh=_sc_mesh,
    out_type=jax.ShapeDtypeStruct((_D, _NIDX), jnp.float32),
    scratch_types=[
        pltpu.VMEM((_BPT,), jnp.int32),
        [pltpu.VMEM((_D, 128), jnp.float32) for _ in range(_CH)],
        pltpu.VMEM((_D, _BPT), jnp.float32),
        pltpu.SemaphoreType.DMA,
    ],
)
def _sc_gather(idx_hbm, table_hbm, out_hbm, idx_v, blks, cols_v, sem):
    wid = lax.axis_index("s") * _NC + lax.axis_index("c")

    @pl.when(wid < _NACT)
    def _():
        base = wid * _BPT
        pltpu.sync_copy(idx_hbm.at[pl.ds(base, _BPT)], idx_v)

        def chunk(ci, carry):
            c = ci * _CH
            v = idx_v[pl.ds(c, _CH)]
            cps = []
            for j in range(_CH):
                blk = lax.shift_right_logical(v[j], 7) * 128
                cps.append(pltpu.async_copy(
                    table_hbm.at[:, pl.ds(pl.multiple_of(blk, 128), 128)],
                    blks[j], sem))
            for j in range(_CH):
                cps[j].wait()
                q = v[j] & 127
                cols_v[pl.ds(0, 16), pl.ds(c + j, 1)] = (
                    blks[j][pl.ds(0, 16), pl.ds(q, 1)])
                cols_v[pl.ds(16, 16), pl.ds(c + j, 1)] = (
                    blks[j][pl.ds(16, 16), pl.ds(q, 1)])
            return carry

        lax.fori_loop(0, _BPT // _CH, chunk, 0)
        pltpu.sync_copy(cols_v, out_hbm.at[:, pl.ds(base, _BPT)])


def _tc_distance_body(h_ref, t_ref, rel_ref, out_ref):
    d = h_ref[...] - t_ref[...]
    acc = jnp.abs(rel_ref[:, 0:1] + d[0:1, :])
    for k in range(1, _D):
        acc = acc + jnp.abs(rel_ref[:, k:k + 1] + d[k:k + 1, :])
    out_ref[...] = -acc


def _tc_distance(rows_t, rel):
    return pl.pallas_call(
        _tc_distance_body,
        grid=(1,),
        in_specs=[
            pl.BlockSpec((_D, _B), lambda i: (0, 0)),
            pl.BlockSpec((_D, _B), lambda i: (0, 1)),
            pl.BlockSpec((_R, _D), lambda i: (0, 0)),
        ],
        out_specs=pl.BlockSpec((_R, _B), lambda i: (0, 0)),
        out_shape=jax.ShapeDtypeStruct((_R, _B), jnp.float32),
    )(rows_t, rows_t, rel)


def kernel(heads, tails, entity_emb, relation_emb):
    idx = jnp.concatenate([heads, tails]).astype(jnp.int32)
    table_t = entity_emb.T
    rows_t = _sc_gather(idx, table_t)
    out_t = _tc_distance(rows_t, relation_emb)
    return out_t.T
